# Initial kernel scaffold; baseline (speedup 1.0000x reference)
import jax
import jax.numpy as jnp
from jax.experimental import pallas as pl


def kernel(x, edge_index, edge_attr, W1, b1, W2, b2):
    raise NotImplementedError("write your pallas kernel here")


# speedup vs baseline: 2.8775x; 2.8775x over previous
import functools

import jax
import jax.numpy as jnp
from jax.experimental import pallas as pl
from jax.experimental.pallas import tpu as pltpu

N_NODES = 10000
NPAD = 10240
RB = 640
GRID = NPAD // RB


def _g_body(x_ref, w_ref, d0_ref, d1_ref, g_ref):
    dis = jax.lax.rsqrt(1.0 + d0_ref[...] + d1_ref[...])
    g_ref[...] = dis * jnp.dot(x_ref[...], w_ref[...],
                               preferred_element_type=jnp.float32)


def _tc_g(xp, W, d0, d1):
    h = W.shape[1]
    return pl.pallas_call(
        _g_body,
        grid=(GRID,),
        in_specs=[
            pl.BlockSpec((RB, xp.shape[1]), lambda i: (i, 0)),
            pl.BlockSpec((W.shape[0], h), lambda i: (0, 0)),
            pl.BlockSpec((RB, 1), lambda i: (i, 0)),
            pl.BlockSpec((RB, 1), lambda i: (i, 0)),
        ],
        out_specs=pl.BlockSpec((RB, h), lambda i: (i, 0)),
        out_shape=jax.ShapeDtypeStruct((NPAD, h), jnp.float32),
    )(xp, W, d0, d1)


def _mid_body(a0_ref, a1_ref, g_ref, d0_ref, d1_ref, w_ref, b_ref, g2_ref):
    dis = jax.lax.rsqrt(1.0 + d0_ref[...] + d1_ref[...])
    z = jax.nn.relu(dis * (a0_ref[...] + a1_ref[...] + g_ref[...]) + b_ref[...])
    g2_ref[...] = dis * jnp.dot(z, w_ref[...],
                                preferred_element_type=jnp.float32)


def _tc_mid(a0, a1, g, d0, d1, W, b):
    h0 = g.shape[1]
    h1 = W.shape[1]
    return pl.pallas_call(
        _mid_body,
        grid=(GRID,),
        in_specs=[
            pl.BlockSpec((RB, h0), lambda i: (i, 0)),
            pl.BlockSpec((RB, h0), lambda i: (i, 0)),
            pl.BlockSpec((RB, h0), lambda i: (i, 0)),
            pl.BlockSpec((RB, 1), lambda i: (i, 0)),
            pl.BlockSpec((RB, 1), lambda i: (i, 0)),
            pl.BlockSpec((h0, h1), lambda i: (0, 0)),
            pl.BlockSpec((1, h0), lambda i: (0, 0)),
        ],
        out_specs=pl.BlockSpec((RB, h1), lambda i: (i, 0)),
        out_shape=jax.ShapeDtypeStruct((NPAD, h1), jnp.float32),
    )(a0, a1, g, d0, d1, W, b)


def _fin_body(a0_ref, a1_ref, g_ref, d0_ref, d1_ref, b_ref, o_ref):
    dis = jax.lax.rsqrt(1.0 + d0_ref[...] + d1_ref[...])
    t = dis * (a0_ref[...] + a1_ref[...] + g_ref[...]) + b_ref[...]
    t = t - jnp.max(t, axis=1, keepdims=True)
    e = jnp.exp(t)
    o_ref[...] = e / jnp.sum(e, axis=1, keepdims=True)


def _tc_fin(a0, a1, g, d0, d1, b):
    h = g.shape[1]
    return pl.pallas_call(
        _fin_body,
        grid=(GRID,),
        in_specs=[
            pl.BlockSpec((RB, h), lambda i: (i, 0)),
            pl.BlockSpec((RB, h), lambda i: (i, 0)),
            pl.BlockSpec((RB, h), lambda i: (i, 0)),
            pl.BlockSpec((RB, 1), lambda i: (i, 0)),
            pl.BlockSpec((RB, 1), lambda i: (i, 0)),
            pl.BlockSpec((1, h), lambda i: (0, 0)),
        ],
        out_specs=pl.BlockSpec((RB, h), lambda i: (i, 0)),
        out_shape=jax.ShapeDtypeStruct((NPAD, h), jnp.float32),
    )(a0, a1, g, d0, d1, b)


def kernel(x, edge_index, edge_attr, W1, b1, W2, b2):
    src = edge_index[0].astype(jnp.int32)
    dst = edge_index[1].astype(jnp.int32)
    ew = edge_attr

    xp = jnp.zeros((NPAD, x.shape[1]), jnp.float32).at[:N_NODES].set(x)

    degp = jnp.zeros((2, NPAD), jnp.float32).at[0, dst].add(ew)
    d0 = degp[0].reshape(NPAD, 1)
    d1 = degp[1].reshape(NPAD, 1)

    g1 = _tc_g(xp, W1, d0, d1)

    a1p = jnp.zeros((2, NPAD, W1.shape[1]), jnp.float32).at[0, dst].add(
        ew[:, None] * g1[src])
    g2 = _tc_mid(a1p[0], a1p[1], g1, d0, d1, W2, b1.reshape(1, -1))

    a2p = jnp.zeros((2, NPAD, W2.shape[1]), jnp.float32).at[0, dst].add(
        ew[:, None] * g2[src])
    out = _tc_fin(a2p[0], a2p[1], g2, d0, d1, b2.reshape(1, -1))
    return out[:N_NODES]


# speedup vs baseline: 10.2034x; 3.5460x over previous
import functools

import jax
import jax.numpy as jnp
from jax import lax
from jax.experimental import pallas as pl
from jax.experimental.pallas import tpu as pltpu
from jax.experimental.pallas import tpu_sc as plsc

N_NODES = 10000
NPAD = 10240
RB = 640
GRID = NPAD // RB
NC, NS = 2, 16
NW = NC * NS
CHUNK = 80
RPT = NPAD // NS

_MESH = plsc.VectorSubcoreMesh(core_axis_name="c", subcore_axis_name="s")


def _sc_deg(dst, ew):
    epw = dst.shape[0] // NW
    nch = epw // CHUNK

    @functools.partial(
        pl.kernel,
        out_type=(jax.ShapeDtypeStruct((NPAD,), jnp.float32),
                  jax.ShapeDtypeStruct((NPAD,), jnp.float32)),
        mesh=_MESH,
        scratch_types=[
            pltpu.VMEM((CHUNK,), jnp.int32),
            pltpu.VMEM((CHUNK,), jnp.float32),
            pltpu.VMEM_SHARED((NPAD,), jnp.float32),
        ],
    )
    def k(dst_hbm, ew_hbm, out0, out1, idx_v, val_v, acc):
        cid = lax.axis_index("c")
        sid = lax.axis_index("s")
        wid = cid * NS + sid
        for c in range(CHUNK // 16):
            val_v[pl.ds(c * 16, 16)] = jnp.zeros((16,), jnp.float32)
        for i in range(RPT // CHUNK):
            pltpu.sync_copy(val_v, acc.at[pl.ds(sid * RPT + i * CHUNK, CHUNK)])
        plsc.subcore_barrier()

        base = wid * epw

        def body(kk, carry):
            off = base + kk * CHUNK
            pltpu.sync_copy(dst_hbm.at[pl.ds(off, CHUNK)], idx_v)
            pltpu.sync_copy(ew_hbm.at[pl.ds(off, CHUNK)], val_v)
            pltpu.sync_copy(val_v, acc.at[idx_v], add=True)
            return carry

        lax.fori_loop(0, nch, body, 0)
        plsc.subcore_barrier()

        @pl.when(cid == 0)
        def _():
            pltpu.sync_copy(acc.at[pl.ds(sid * RPT, RPT)],
                            out0.at[pl.ds(sid * RPT, RPT)])

        @pl.when(cid == 1)
        def _():
            pltpu.sync_copy(acc.at[pl.ds(sid * RPT, RPT)],
                            out1.at[pl.ds(sid * RPT, RPT)])

    return k(dst, ew)


def _sc_msg(src, dst, ew, g):
    epw = src.shape[0] // NW
    nch = epw // CHUNK
    h = g.shape[1]

    @functools.partial(
        pl.kernel,
        out_type=(jax.ShapeDtypeStruct((NPAD, h), jnp.float32),
                  jax.ShapeDtypeStruct((NPAD, h), jnp.float32)),
        mesh=_MESH,
        scratch_types=[
            pltpu.VMEM((CHUNK,), jnp.int32),
            pltpu.VMEM((CHUNK,), jnp.int32),
            pltpu.VMEM((CHUNK,), jnp.float32),
            pltpu.VMEM((CHUNK, h), jnp.float32),
            pltpu.VMEM_SHARED((NPAD, h), jnp.float32),
            pltpu.SemaphoreType.DMA,
        ],
        compiler_params=pltpu.CompilerParams(use_tc_tiling_on_sc=False),
    )
    def k(src_hbm, dst_hbm, ew_hbm, g_hbm, out0, out1,
          sidx_v, didx_v, ew_v, rows_v, acc, sem):
        cid = lax.axis_index("c")
        sid = lax.axis_index("s")
        wid = cid * NS + sid

        def zbody(j, carry):
            for c in range(h // 16):
                rows_v[j, pl.ds(c * 16, 16)] = jnp.zeros((16,), jnp.float32)
            return carry

        lax.fori_loop(0, CHUNK, zbody, 0)
        for i in range(RPT // CHUNK):
            pltpu.sync_copy(
                rows_v, acc.at[pl.ds(sid * RPT + i * CHUNK, CHUNK), :])
        plsc.subcore_barrier()

        base = wid * epw

        def body(kk, carry):
            off = base + kk * CHUNK
            pltpu.sync_copy(src_hbm.at[pl.ds(off, CHUNK)], sidx_v)
            pltpu.sync_copy(dst_hbm.at[pl.ds(off, CHUNK)], didx_v)
            pltpu.sync_copy(ew_hbm.at[pl.ds(off, CHUNK)], ew_v)
            pltpu.async_copy(g_hbm.at[sidx_v], rows_v, sem).wait()

            for gg in range(CHUNK // 16):
                ew_g = ew_v[pl.ds(gg * 16, 16)]

                def scale(j2, c2, ew_g=ew_g, gg=gg):
                    w = lax.gather(
                        ew_g, jnp.full((16, 1), j2, jnp.int32),
                        lax.GatherDimensionNumbers(
                            offset_dims=(), collapsed_slice_dims=(0,),
                            start_index_map=(0,)),
                        (1,), mode=lax.GatherScatterMode.PROMISE_IN_BOUNDS)
                    j = gg * 16 + j2
                    for c in range(h // 16):
                        sl = pl.ds(c * 16, 16)
                        rows_v[j, sl] = rows_v[j, sl] * w
                    return c2

                lax.fori_loop(0, 16, scale, 0)
            pltpu.sync_copy(rows_v, acc.at[didx_v], add=True)
            return carry

        lax.fori_loop(0, nch, body, 0)
        plsc.subcore_barrier()

        @pl.when(cid == 0)
        def _():
            pltpu.sync_copy(acc.at[pl.ds(sid * RPT, RPT), :],
                            out0.at[pl.ds(sid * RPT, RPT), :])

        @pl.when(cid == 1)
        def _():
            pltpu.sync_copy(acc.at[pl.ds(sid * RPT, RPT), :],
                            out1.at[pl.ds(sid * RPT, RPT), :])

    return k(src, dst, ew, g)


def _g_body(x_ref, w_ref, d0_ref, d1_ref, g_ref):
    dis = jax.lax.rsqrt(1.0 + d0_ref[...] + d1_ref[...])
    g_ref[...] = dis * jnp.dot(x_ref[...], w_ref[...],
                               preferred_element_type=jnp.float32)


def _tc_g(xp, W, d0, d1):
    h = W.shape[1]
    return pl.pallas_call(
        _g_body,
        grid=(GRID,),
        in_specs=[
            pl.BlockSpec((RB, xp.shape[1]), lambda i: (i, 0)),
            pl.BlockSpec((W.shape[0], h), lambda i: (0, 0)),
            pl.BlockSpec((RB, 1), lambda i: (i, 0)),
            pl.BlockSpec((RB, 1), lambda i: (i, 0)),
        ],
        out_specs=pl.BlockSpec((RB, h), lambda i: (i, 0)),
        out_shape=jax.ShapeDtypeStruct((NPAD, h), jnp.float32),
    )(xp, W, d0, d1)


def _mid_body(a0_ref, a1_ref, g_ref, d0_ref, d1_ref, w_ref, b_ref, g2_ref):
    dis = jax.lax.rsqrt(1.0 + d0_ref[...] + d1_ref[...])
    z = jax.nn.relu(dis * (a0_ref[...] + a1_ref[...] + g_ref[...]) + b_ref[...])
    g2_ref[...] = dis * jnp.dot(z, w_ref[...],
                                preferred_element_type=jnp.float32)


def _tc_mid(a0, a1, g, d0, d1, W, b):
    h0 = g.shape[1]
    h1 = W.shape[1]
    return pl.pallas_call(
        _mid_body,
        grid=(GRID,),
        in_specs=[
            pl.BlockSpec((RB, h0), lambda i: (i, 0)),
            pl.BlockSpec((RB, h0), lambda i: (i, 0)),
            pl.BlockSpec((RB, h0), lambda i: (i, 0)),
            pl.BlockSpec((RB, 1), lambda i: (i, 0)),
            pl.BlockSpec((RB, 1), lambda i: (i, 0)),
            pl.BlockSpec((h0, h1), lambda i: (0, 0)),
            pl.BlockSpec((1, h0), lambda i: (0, 0)),
        ],
        out_specs=pl.BlockSpec((RB, h1), lambda i: (i, 0)),
        out_shape=jax.ShapeDtypeStruct((NPAD, h1), jnp.float32),
    )(a0, a1, g, d0, d1, W, b)


def _fin_body(a0_ref, a1_ref, g_ref, d0_ref, d1_ref, b_ref, o_ref):
    dis = jax.lax.rsqrt(1.0 + d0_ref[...] + d1_ref[...])
    t = dis * (a0_ref[...] + a1_ref[...] + g_ref[...]) + b_ref[...]
    t = t - jnp.max(t, axis=1, keepdims=True)
    e = jnp.exp(t)
    o_ref[...] = e / jnp.sum(e, axis=1, keepdims=True)


def _tc_fin(a0, a1, g, d0, d1, b):
    h = g.shape[1]
    return pl.pallas_call(
        _fin_body,
        grid=(GRID,),
        in_specs=[
            pl.BlockSpec((RB, h), lambda i: (i, 0)),
            pl.BlockSpec((RB, h), lambda i: (i, 0)),
            pl.BlockSpec((RB, h), lambda i: (i, 0)),
            pl.BlockSpec((RB, 1), lambda i: (i, 0)),
            pl.BlockSpec((RB, 1), lambda i: (i, 0)),
            pl.BlockSpec((1, h), lambda i: (0, 0)),
        ],
        out_specs=pl.BlockSpec((RB, h), lambda i: (i, 0)),
        out_shape=jax.ShapeDtypeStruct((NPAD, h), jnp.float32),
    )(a0, a1, g, d0, d1, b)


def kernel(x, edge_index, edge_attr, W1, b1, W2, b2):
    src = edge_index[0].astype(jnp.int32)
    dst = edge_index[1].astype(jnp.int32)
    ew = edge_attr

    xp = jnp.zeros((NPAD, x.shape[1]), jnp.float32).at[:N_NODES].set(x)

    deg0, deg1 = _sc_deg(dst, ew)
    d0 = deg0.reshape(NPAD, 1)
    d1 = deg1.reshape(NPAD, 1)

    g1 = _tc_g(xp, W1, d0, d1)
    a1_0, a1_1 = _sc_msg(src, dst, ew, g1)
    g2 = _tc_mid(a1_0, a1_1, g1, d0, d1, W2, b1.reshape(1, -1))
    a2_0, a2_1 = _sc_msg(src, dst, ew, g2)
    out = _tc_fin(a2_0, a2_1, g2, d0, d1, b2.reshape(1, -1))
    return out[:N_NODES]


# speedup vs baseline: 12.8135x; 1.2558x over previous
import functools

import jax
import jax.numpy as jnp
from jax import lax
from jax.experimental import pallas as pl
from jax.experimental.pallas import tpu as pltpu
from jax.experimental.pallas import tpu_sc as plsc

N_NODES = 10000
NPAD = 10240
RB = 640
GRID = NPAD // RB
NC, NS = 2, 16
NW = NC * NS
CHUNK = 80
RPT = NPAD // NS

_MESH = plsc.VectorSubcoreMesh(core_axis_name="c", subcore_axis_name="s")


CH = 128
NCHUNK = 80
TOTC = NCHUNK + 2


def _sc_deg(dstp, ewp):
    @functools.partial(
        pl.kernel,
        out_type=(jax.ShapeDtypeStruct((NPAD,), jnp.float32),
                  jax.ShapeDtypeStruct((NPAD,), jnp.float32)),
        mesh=_MESH,
        scratch_types=[
            pltpu.VMEM((TOTC, CH), jnp.int32),
            pltpu.VMEM((TOTC, CH), jnp.float32),
            pltpu.VMEM((CH,), jnp.float32),
            pltpu.VMEM_SHARED((NPAD,), jnp.float32),
        ],
        compiler_params=pltpu.CompilerParams(use_tc_tiling_on_sc=False),
    )
    def k(dst_hbm, ew_hbm, out0, out1, dst_big, ew_big, zv, acc):
        cid = lax.axis_index("c")
        sid = lax.axis_index("s")
        wid = cid * NS + sid
        pltpu.sync_copy(dst_hbm.at[wid], dst_big)
        pltpu.sync_copy(ew_hbm.at[wid], ew_big)
        for c in range(CH // 16):
            zv[pl.ds(c * 16, 16)] = jnp.zeros((16,), jnp.float32)
        for i in range(RPT // CH):
            pltpu.sync_copy(zv, acc.at[pl.ds(sid * RPT + i * CH, CH)])
        plsc.subcore_barrier()

        def body(kk, carry):
            pltpu.sync_copy(ew_big.at[kk], acc.at[dst_big.at[kk]], add=True)
            return carry

        lax.fori_loop(0, NCHUNK, body, 0)
        plsc.subcore_barrier()

        @pl.when(cid == 0)
        def _():
            pltpu.sync_copy(acc.at[pl.ds(sid * RPT, RPT)],
                            out0.at[pl.ds(sid * RPT, RPT)])

        @pl.when(cid == 1)
        def _():
            pltpu.sync_copy(acc.at[pl.ds(sid * RPT, RPT)],
                            out1.at[pl.ds(sid * RPT, RPT)])

    return k(dstp, ewp)


PCH = 80


def _sc_msg(srcp, dstp, ewp, g, phases, colsplit):
    h = 64

    @functools.partial(
        pl.kernel,
        out_type=(jax.ShapeDtypeStruct((NPAD, h), jnp.float32),
                  jax.ShapeDtypeStruct((NPAD, h), jnp.float32)),
        mesh=_MESH,
        scratch_types=[
            pltpu.VMEM((PCH + 2, CH), jnp.int32),
            pltpu.VMEM((PCH + 2, CH), jnp.int32),
            pltpu.VMEM((PCH + 2, CH), jnp.float32),
            pltpu.VMEM((CH, h), jnp.float32),
            pltpu.VMEM((CH, h), jnp.float32),
            pltpu.VMEM((CH, h), jnp.float32),
            pltpu.VMEM((CH, h), jnp.float32),
            pltpu.VMEM_SHARED((NPAD, h), jnp.float32),
            pltpu.SemaphoreType.DMA,
            pltpu.SemaphoreType.DMA,
            pltpu.SemaphoreType.DMA,
            pltpu.SemaphoreType.DMA,
        ],
        compiler_params=pltpu.CompilerParams(use_tc_tiling_on_sc=False),
    )
    def k(src_hbm, dst_hbm, ew_hbm, g_hbm, out0, out1,
          src_big, dst_big, ew_big, rows0, rows1, msg0, msg1, acc,
          sg0, sg1, ss0, ss1):
        cid = lax.axis_index("c")
        sid = lax.axis_index("s")
        eb = sid if colsplit else cid * NS + sid

        rows = (rows0, rows1)
        msgs = (msg0, msg1)
        sgs = (sg0, sg1)
        sss = (ss0, ss1)

        def stage(p):
            pltpu.sync_copy(src_hbm.at[eb, pl.ds(p * PCH, PCH + 2)], src_big)
            pltpu.sync_copy(dst_hbm.at[eb, pl.ds(p * PCH, PCH + 2)], dst_big)
            pltpu.sync_copy(ew_hbm.at[eb, pl.ds(p * PCH, PCH + 2)], ew_big)
            if colsplit:
                off = jnp.zeros((16,), jnp.int32) + cid * NPAD

                def obody(r, carry):
                    for c in range(CH // 16):
                        sl = pl.ds(c * 16, 16)
                        src_big[r, sl] = src_big[r, sl] + off
                    return carry

                lax.fori_loop(0, PCH + 2, obody, 0)

        stage(0)
        pltpu.async_copy(g_hbm.at[src_big.at[0]], rows0, sg0)
        pltpu.async_copy(g_hbm.at[src_big.at[1]], rows1, sg1)

        def zbody(j, carry):
            for c in range(h // 16):
                msg0[j, pl.ds(c * 16, 16)] = jnp.zeros((16,), jnp.float32)
            return carry

        lax.fori_loop(0, CH, zbody, 0)
        for i in range(RPT // CH):
            pltpu.sync_copy(msg0, acc.at[pl.ds(sid * RPT + i * CH, CH), :])
        plsc.subcore_barrier()

        for p in range(phases):
            if p > 0:
                stage(p)
                pltpu.async_copy(g_hbm.at[src_big.at[0]], rows0, sg0)
                pltpu.async_copy(g_hbm.at[src_big.at[1]], rows1, sg1)

            def pair(kk, carry):
                for b in range(2):
                    kchunk = 2 * kk + b
                    pltpu.make_async_copy(g_hbm.at[src_big.at[kchunk]],
                                          rows[b], sgs[b]).wait()

                    @pl.when(kk >= 1)
                    def _():
                        pltpu.make_async_copy(msgs[b],
                                              acc.at[dst_big.at[kchunk]],
                                              sss[b]).wait()

                    def grp(gg, c2, b=b, kchunk=kchunk):
                        ew_g = ew_big[kchunk, pl.ds(gg * 16, 16)]

                        def scale(j2, c3, ew_g=ew_g, gg=gg, b=b):
                            w = lax.gather(
                                ew_g, jnp.full((16, 1), j2, jnp.int32),
                                lax.GatherDimensionNumbers(
                                    offset_dims=(), collapsed_slice_dims=(0,),
                                    start_index_map=(0,)),
                                (1,),
                                mode=lax.GatherScatterMode.PROMISE_IN_BOUNDS)
                            j = gg * 16 + j2
                            for c in range(h // 16):
                                sl = pl.ds(c * 16, 16)
                                msgs[b][j, sl] = rows[b][j, sl] * w
                            return c3

                        lax.fori_loop(0, 16, scale, 0)
                        return c2

                    lax.fori_loop(0, CH // 16, grp, 0)
                    pltpu.async_copy(g_hbm.at[src_big.at[kchunk + 2]],
                                     rows[b], sgs[b])
                    pltpu.async_copy(msgs[b], acc.at[dst_big.at[kchunk]],
                                     sss[b], add=True)
                return carry

            lax.fori_loop(0, PCH // 2, pair, 0)

            for b in range(2):
                pltpu.make_async_copy(msgs[b],
                                      acc.at[dst_big.at[PCH - 2 + b]],
                                      sss[b]).wait()
                pltpu.make_async_copy(g_hbm.at[src_big.at[PCH + b]],
                                      rows[b], sgs[b]).wait()

        plsc.subcore_barrier()

        @pl.when(cid == 0)
        def _():
            pltpu.sync_copy(acc.at[pl.ds(sid * RPT, RPT), :],
                            out0.at[pl.ds(sid * RPT, RPT), :])

        @pl.when(cid == 1)
        def _():
            pltpu.sync_copy(acc.at[pl.ds(sid * RPT, RPT), :],
                            out1.at[pl.ds(sid * RPT, RPT), :])

    return k(srcp, dstp, ewp, g)


def _g_body(x_ref, w_ref, d0_ref, d1_ref, g_ref):
    dis = jax.lax.rsqrt(1.0 + d0_ref[...] + d1_ref[...])
    g_ref[...] = dis * jnp.dot(x_ref[...], w_ref[...],
                               preferred_element_type=jnp.float32)


def _tc_g(xp, W, d0, d1):
    h = W.shape[1]
    return pl.pallas_call(
        _g_body,
        grid=(GRID,),
        in_specs=[
            pl.BlockSpec((RB, xp.shape[1]), lambda i: (i, 0)),
            pl.BlockSpec((W.shape[0], h), lambda i: (0, 0)),
            pl.BlockSpec((RB, 1), lambda i: (i, 0)),
            pl.BlockSpec((RB, 1), lambda i: (i, 0)),
        ],
        out_specs=pl.BlockSpec((RB, h), lambda i: (i, 0)),
        out_shape=jax.ShapeDtypeStruct((NPAD, h), jnp.float32),
    )(xp, W, d0, d1)


def _mid_body(a0_ref, a1_ref, g_ref, d0_ref, d1_ref, w_ref, b_ref, g2_ref):
    dis = jax.lax.rsqrt(1.0 + d0_ref[...] + d1_ref[...])
    a = jnp.concatenate([a0_ref[...], a1_ref[...]], axis=1)
    z = jax.nn.relu(dis * (a + g_ref[...]) + b_ref[...])
    g2_ref[...] = dis * jnp.dot(z, w_ref[...],
                                preferred_element_type=jnp.float32)


def _tc_mid(a0, a1, g, d0, d1, W, b):
    h0 = g.shape[1]
    h1 = W.shape[1]
    return pl.pallas_call(
        _mid_body,
        grid=(GRID,),
        in_specs=[
            pl.BlockSpec((RB, h0 // 2), lambda i: (i, 0)),
            pl.BlockSpec((RB, h0 // 2), lambda i: (i, 0)),
            pl.BlockSpec((RB, h0), lambda i: (i, 0)),
            pl.BlockSpec((RB, 1), lambda i: (i, 0)),
            pl.BlockSpec((RB, 1), lambda i: (i, 0)),
            pl.BlockSpec((h0, h1), lambda i: (0, 0)),
            pl.BlockSpec((1, h0), lambda i: (0, 0)),
        ],
        out_specs=pl.BlockSpec((RB, h1), lambda i: (i, 0)),
        out_shape=jax.ShapeDtypeStruct((NPAD, h1), jnp.float32),
    )(a0, a1, g, d0, d1, W, b)


def _fin_body(a0_ref, a1_ref, g_ref, d0_ref, d1_ref, b_ref, o_ref):
    dis = jax.lax.rsqrt(1.0 + d0_ref[...] + d1_ref[...])
    t = dis * (a0_ref[...] + a1_ref[...] + g_ref[...]) + b_ref[...]
    t = t - jnp.max(t, axis=1, keepdims=True)
    e = jnp.exp(t)
    o_ref[...] = e / jnp.sum(e, axis=1, keepdims=True)


def _tc_fin(a0, a1, g, d0, d1, b):
    h = g.shape[1]
    return pl.pallas_call(
        _fin_body,
        grid=(GRID,),
        in_specs=[
            pl.BlockSpec((RB, h), lambda i: (i, 0)),
            pl.BlockSpec((RB, h), lambda i: (i, 0)),
            pl.BlockSpec((RB, h), lambda i: (i, 0)),
            pl.BlockSpec((RB, 1), lambda i: (i, 0)),
            pl.BlockSpec((RB, 1), lambda i: (i, 0)),
            pl.BlockSpec((1, h), lambda i: (0, 0)),
        ],
        out_specs=pl.BlockSpec((RB, h), lambda i: (i, 0)),
        out_shape=jax.ShapeDtypeStruct((NPAD, h), jnp.float32),
    )(a0, a1, g, d0, d1, b)


def kernel(x, edge_index, edge_attr, W1, b1, W2, b2):
    src = edge_index[0].astype(jnp.int32)
    dst = edge_index[1].astype(jnp.int32)
    ew = edge_attr
    e_tot = src.shape[0]

    def staged(nblk, tot):
        ept = e_tot // nblk
        pad = tot * CH - ept
        pidx = ((jnp.arange(pad, dtype=jnp.int32) % (NPAD - N_NODES))
                + N_NODES)
        pidx = jnp.broadcast_to(pidx, (nblk, pad))
        sp = jnp.concatenate([src.reshape(nblk, ept), pidx],
                             axis=1).reshape(nblk, tot, CH)
        dp = jnp.concatenate([dst.reshape(nblk, ept), pidx],
                             axis=1).reshape(nblk, tot, CH)
        wp = jnp.concatenate([ew.reshape(nblk, ept),
                              jnp.zeros((nblk, pad), jnp.float32)],
                             axis=1).reshape(nblk, tot, CH)
        return sp, dp, wp

    srcp1, dstp1, ewp1 = staged(NS, 2 * PCH + 2)
    srcp2, dstp2, ewp2 = staged(NW, PCH + 2)

    xp = jnp.zeros((NPAD, x.shape[1]), jnp.float32).at[:N_NODES].set(x)

    deg0, deg1 = _sc_deg(dstp2, ewp2)
    d0 = deg0.reshape(NPAD, 1)
    d1 = deg1.reshape(NPAD, 1)

    g1 = _tc_g(xp, W1, d0, d1)
    gstk = jnp.concatenate([g1[:, :64], g1[:, 64:]], axis=0)
    a1_lo, a1_hi = _sc_msg(srcp1, dstp1, ewp1, gstk, 2, True)
    g2 = _tc_mid(a1_lo, a1_hi, g1, d0, d1, W2, b1.reshape(1, -1))
    a2_0, a2_1 = _sc_msg(srcp2, dstp2, ewp2, g2, 1, False)
    out = _tc_fin(a2_0, a2_1, g2, d0, d1, b2.reshape(1, -1))
    return out[:N_NODES]


# speedup vs baseline: 27.0060x; 2.1076x over previous
import functools

import jax
import jax.numpy as jnp
from jax import lax
from jax.experimental import pallas as pl
from jax.experimental.pallas import tpu as pltpu
from jax.experimental.pallas import tpu_sc as plsc

N_NODES = 10000
NPAD = 10240
RB = 640
GRID = NPAD // RB
NC, NS = 2, 16
NW = NC * NS
CHUNK = 80
RPT = NPAD // NS

_MESH = plsc.VectorSubcoreMesh(core_axis_name="c", subcore_axis_name="s")


CH = 128
NCHUNK = 80
TOTC = NCHUNK + 2


def _sc_deg(dstp, ewp):
    @functools.partial(
        pl.kernel,
        out_type=(jax.ShapeDtypeStruct((NPAD,), jnp.float32),
                  jax.ShapeDtypeStruct((NPAD,), jnp.float32)),
        mesh=_MESH,
        scratch_types=[
            pltpu.VMEM((TOTC, CH), jnp.int32),
            pltpu.VMEM((TOTC, CH), jnp.float32),
            pltpu.VMEM((CH,), jnp.float32),
            pltpu.VMEM_SHARED((NPAD,), jnp.float32),
        ],
        compiler_params=pltpu.CompilerParams(use_tc_tiling_on_sc=False),
    )
    def k(dst_hbm, ew_hbm, out0, out1, dst_big, ew_big, zv, acc):
        cid = lax.axis_index("c")
        sid = lax.axis_index("s")
        wid = cid * NS + sid
        pltpu.sync_copy(dst_hbm.at[wid], dst_big)
        pltpu.sync_copy(ew_hbm.at[wid], ew_big)
        for c in range(CH // 16):
            zv[pl.ds(c * 16, 16)] = jnp.zeros((16,), jnp.float32)
        for i in range(RPT // CH):
            pltpu.sync_copy(zv, acc.at[pl.ds(sid * RPT + i * CH, CH)])
        plsc.subcore_barrier()

        def body(kk, carry):
            pltpu.sync_copy(ew_big.at[kk], acc.at[dst_big.at[kk]], add=True)
            return carry

        lax.fori_loop(0, NCHUNK, body, 0)
        plsc.subcore_barrier()

        @pl.when(cid == 0)
        def _():
            pltpu.sync_copy(acc.at[pl.ds(sid * RPT, RPT)],
                            out0.at[pl.ds(sid * RPT, RPT)])

        @pl.when(cid == 1)
        def _():
            pltpu.sync_copy(acc.at[pl.ds(sid * RPT, RPT)],
                            out1.at[pl.ds(sid * RPT, RPT)])

    return k(dstp, ewp)


PCH = 80


def _sc_msg(srcp, dstp, ewp, g, phases, colsplit):
    h = 64

    @functools.partial(
        pl.kernel,
        out_type=(jax.ShapeDtypeStruct((NPAD, h), jnp.float32),
                  jax.ShapeDtypeStruct((NPAD, h), jnp.float32)),
        mesh=_MESH,
        scratch_types=[
            pltpu.VMEM((PCH + 2, CH), jnp.int32),
            pltpu.VMEM((PCH + 2, CH), jnp.int32),
            pltpu.VMEM((PCH + 2, CH), jnp.float32),
            pltpu.VMEM((CH, h), jnp.float32),
            pltpu.VMEM((CH, h), jnp.float32),
            pltpu.VMEM((CH, h), jnp.float32),
            pltpu.VMEM((CH, h), jnp.float32),
            pltpu.VMEM_SHARED((NPAD, h), jnp.float32),
            pltpu.SemaphoreType.DMA,
            pltpu.SemaphoreType.DMA,
            pltpu.SemaphoreType.DMA,
            pltpu.SemaphoreType.DMA,
        ],
        compiler_params=pltpu.CompilerParams(use_tc_tiling_on_sc=False),
    )
    def k(src_hbm, dst_hbm, ew_hbm, g_hbm, out0, out1,
          src_big, dst_big, ew_big, rows0, rows1, msg0, msg1, acc,
          sg0, sg1, ss0, ss1):
        cid = lax.axis_index("c")
        sid = lax.axis_index("s")
        eb = sid if colsplit else cid * NS + sid

        rows = (rows0, rows1)
        msgs = (msg0, msg1)
        sgs = (sg0, sg1)
        sss = (ss0, ss1)

        def stage(p):
            pltpu.sync_copy(src_hbm.at[eb, pl.ds(p * PCH, PCH + 2)], src_big)
            pltpu.sync_copy(dst_hbm.at[eb, pl.ds(p * PCH, PCH + 2)], dst_big)
            pltpu.sync_copy(ew_hbm.at[eb, pl.ds(p * PCH, PCH + 2)], ew_big)
            if colsplit:
                off = jnp.zeros((16,), jnp.int32) + cid * NPAD

                def obody(r, carry):
                    for c in range(CH // 16):
                        sl = pl.ds(c * 16, 16)
                        src_big[r, sl] = src_big[r, sl] + off
                    return carry

                lax.fori_loop(0, PCH + 2, obody, 0)

        stage(0)
        pltpu.async_copy(g_hbm.at[src_big.at[0]], rows0, sg0)
        pltpu.async_copy(g_hbm.at[src_big.at[1]], rows1, sg1)

        def zbody(j, carry):
            for c in range(h // 16):
                msg0[j, pl.ds(c * 16, 16)] = jnp.zeros((16,), jnp.float32)
            return carry

        lax.fori_loop(0, CH, zbody, 0)
        for i in range(RPT // CH):
            pltpu.sync_copy(msg0, acc.at[pl.ds(sid * RPT + i * CH, CH), :])
        plsc.subcore_barrier()

        for p in range(phases):
            if p > 0:
                stage(p)
                pltpu.async_copy(g_hbm.at[src_big.at[0]], rows0, sg0)
                pltpu.async_copy(g_hbm.at[src_big.at[1]], rows1, sg1)

            def pair(kk, carry):
                for b in range(2):
                    kchunk = 2 * kk + b
                    pltpu.make_async_copy(g_hbm.at[src_big.at[kchunk]],
                                          rows[b], sgs[b]).wait()

                    @pl.when(kk >= 1)
                    def _():
                        pltpu.make_async_copy(msgs[b],
                                              acc.at[dst_big.at[kchunk]],
                                              sss[b]).wait()

                    def grp(gg, c2, b=b, kchunk=kchunk):
                        ew_g = ew_big[kchunk, pl.ds(gg * 16, 16)]
                        jb = gg * 16
                        for j2 in range(16):
                            w = lax.gather(
                                ew_g, jnp.full((16, 1), j2, jnp.int32),
                                lax.GatherDimensionNumbers(
                                    offset_dims=(), collapsed_slice_dims=(0,),
                                    start_index_map=(0,)),
                                (1,),
                                mode=lax.GatherScatterMode.PROMISE_IN_BOUNDS)
                            for c in range(h // 16):
                                sl = pl.ds(c * 16, 16)
                                msgs[b][jb + j2, sl] = rows[b][jb + j2, sl] * w
                        return c2

                    lax.fori_loop(0, CH // 16, grp, 0)
                    pltpu.async_copy(g_hbm.at[src_big.at[kchunk + 2]],
                                     rows[b], sgs[b])
                    pltpu.async_copy(msgs[b], acc.at[dst_big.at[kchunk]],
                                     sss[b], add=True)
                return carry

            lax.fori_loop(0, PCH // 2, pair, 0)

            for b in range(2):
                pltpu.make_async_copy(msgs[b],
                                      acc.at[dst_big.at[PCH - 2 + b]],
                                      sss[b]).wait()
                pltpu.make_async_copy(g_hbm.at[src_big.at[PCH + b]],
                                      rows[b], sgs[b]).wait()

        plsc.subcore_barrier()

        @pl.when(cid == 0)
        def _():
            pltpu.sync_copy(acc.at[pl.ds(sid * RPT, RPT), :],
                            out0.at[pl.ds(sid * RPT, RPT), :])

        @pl.when(cid == 1)
        def _():
            pltpu.sync_copy(acc.at[pl.ds(sid * RPT, RPT), :],
                            out1.at[pl.ds(sid * RPT, RPT), :])

    return k(srcp, dstp, ewp, g)


def _g_body(x_ref, w_ref, d0_ref, d1_ref, g_ref):
    dis = jax.lax.rsqrt(1.0 + d0_ref[...] + d1_ref[...])
    g_ref[...] = dis * jnp.dot(x_ref[...], w_ref[...],
                               preferred_element_type=jnp.float32)


def _tc_g(xp, W, d0, d1):
    h = W.shape[1]
    return pl.pallas_call(
        _g_body,
        grid=(GRID,),
        in_specs=[
            pl.BlockSpec((RB, xp.shape[1]), lambda i: (i, 0)),
            pl.BlockSpec((W.shape[0], h), lambda i: (0, 0)),
            pl.BlockSpec((RB, 1), lambda i: (i, 0)),
            pl.BlockSpec((RB, 1), lambda i: (i, 0)),
        ],
        out_specs=pl.BlockSpec((RB, h), lambda i: (i, 0)),
        out_shape=jax.ShapeDtypeStruct((NPAD, h), jnp.float32),
    )(xp, W, d0, d1)


def _mid_body(a0_ref, a1_ref, g_ref, d0_ref, d1_ref, w_ref, b_ref, g2_ref):
    dis = jax.lax.rsqrt(1.0 + d0_ref[...] + d1_ref[...])
    a = jnp.concatenate([a0_ref[...], a1_ref[...]], axis=1)
    z = jax.nn.relu(dis * (a + g_ref[...]) + b_ref[...])
    g2_ref[...] = dis * jnp.dot(z, w_ref[...],
                                preferred_element_type=jnp.float32)


def _tc_mid(a0, a1, g, d0, d1, W, b):
    h0 = g.shape[1]
    h1 = W.shape[1]
    return pl.pallas_call(
        _mid_body,
        grid=(GRID,),
        in_specs=[
            pl.BlockSpec((RB, h0 // 2), lambda i: (i, 0)),
            pl.BlockSpec((RB, h0 // 2), lambda i: (i, 0)),
            pl.BlockSpec((RB, h0), lambda i: (i, 0)),
            pl.BlockSpec((RB, 1), lambda i: (i, 0)),
            pl.BlockSpec((RB, 1), lambda i: (i, 0)),
            pl.BlockSpec((h0, h1), lambda i: (0, 0)),
            pl.BlockSpec((1, h0), lambda i: (0, 0)),
        ],
        out_specs=pl.BlockSpec((RB, h1), lambda i: (i, 0)),
        out_shape=jax.ShapeDtypeStruct((NPAD, h1), jnp.float32),
    )(a0, a1, g, d0, d1, W, b)


def _fin_body(a0_ref, a1_ref, g_ref, d0_ref, d1_ref, b_ref, o_ref):
    dis = jax.lax.rsqrt(1.0 + d0_ref[...] + d1_ref[...])
    t = dis * (a0_ref[...] + a1_ref[...] + g_ref[...]) + b_ref[...]
    t = t - jnp.max(t, axis=1, keepdims=True)
    e = jnp.exp(t)
    o_ref[...] = e / jnp.sum(e, axis=1, keepdims=True)


def _tc_fin(a0, a1, g, d0, d1, b):
    h = g.shape[1]
    return pl.pallas_call(
        _fin_body,
        grid=(GRID,),
        in_specs=[
            pl.BlockSpec((RB, h), lambda i: (i, 0)),
            pl.BlockSpec((RB, h), lambda i: (i, 0)),
            pl.BlockSpec((RB, h), lambda i: (i, 0)),
            pl.BlockSpec((RB, 1), lambda i: (i, 0)),
            pl.BlockSpec((RB, 1), lambda i: (i, 0)),
            pl.BlockSpec((1, h), lambda i: (0, 0)),
        ],
        out_specs=pl.BlockSpec((RB, h), lambda i: (i, 0)),
        out_shape=jax.ShapeDtypeStruct((NPAD, h), jnp.float32),
    )(a0, a1, g, d0, d1, b)


def kernel(x, edge_index, edge_attr, W1, b1, W2, b2):
    src = edge_index[0].astype(jnp.int32)
    dst = edge_index[1].astype(jnp.int32)
    ew = edge_attr
    e_tot = src.shape[0]

    def staged(nblk, tot):
        ept = e_tot // nblk
        pad = tot * CH - ept
        pidx = ((jnp.arange(pad, dtype=jnp.int32) % (NPAD - N_NODES))
                + N_NODES)
        pidx = jnp.broadcast_to(pidx, (nblk, pad))
        sp = jnp.concatenate([src.reshape(nblk, ept), pidx],
                             axis=1).reshape(nblk, tot, CH)
        dp = jnp.concatenate([dst.reshape(nblk, ept), pidx],
                             axis=1).reshape(nblk, tot, CH)
        wp = jnp.concatenate([ew.reshape(nblk, ept),
                              jnp.zeros((nblk, pad), jnp.float32)],
                             axis=1).reshape(nblk, tot, CH)
        return sp, dp, wp

    srcp1, dstp1, ewp1 = staged(NS, 2 * PCH + 2)
    srcp2, dstp2, ewp2 = staged(NW, PCH + 2)

    xp = jnp.zeros((NPAD, x.shape[1]), jnp.float32).at[:N_NODES].set(x)

    deg0, deg1 = _sc_deg(dstp2, ewp2)
    d0 = deg0.reshape(NPAD, 1)
    d1 = deg1.reshape(NPAD, 1)

    g1 = _tc_g(xp, W1, d0, d1)
    gstk = jnp.concatenate([g1[:, :64], g1[:, 64:]], axis=0)
    a1_lo, a1_hi = _sc_msg(srcp1, dstp1, ewp1, gstk, 2, True)
    g2 = _tc_mid(a1_lo, a1_hi, g1, d0, d1, W2, b1.reshape(1, -1))
    a2_0, a2_1 = _sc_msg(srcp2, dstp2, ewp2, g2, 1, False)
    out = _tc_fin(a2_0, a2_1, g2, d0, d1, b2.reshape(1, -1))
    return out[:N_NODES]
